# Initial kernel scaffold; baseline (speedup 1.0000x reference)
import jax
import jax.numpy as jnp
from jax.experimental import pallas as pl


def kernel(features, boxes_xywh, obj_classes, w_emb, b_emb, w_obj, b_obj, w_attr, b_attr, w_bbox, b_bbox, cls_table, w_s1, b_s1, w_s2, b_s2, w_f1, b_f1, w_f2, b_f2, w_rel, b_rel):
    raise NotImplementedError("write your pallas kernel here")


# speedup vs baseline: 4.9502x; 4.9502x over previous
import numpy as np
import jax
import jax.numpy as jnp
from jax.experimental import pallas as pl
from jax.experimental.pallas import tpu as pltpu

B, C, H, W = 8, 256, 128, 128
N = 64
R = 7
E = 512
HID = 256
NOBJ, NREL, NATTR = 150, 50, 200

CB = 64
NB = C // CB


def _roi_head_body(ftT_ref, boxes_ref, wp_ref, b_emb_ref,
                   w_obj_ref, b_obj_ref, w_attr_ref, b_attr_ref,
                   w_bbox_ref, b_bbox_ref,
                   obj_out, attr_out, bbox_out, acc_ref):
    k = pl.program_id(1)
    bx = boxes_ref[0]
    cx, cy = bx[:, 0:1], bx[:, 1:2]
    w, h = bx[:, 2:3], bx[:, 3:4]
    x1 = jnp.floor(jnp.clip((cx - w * 0.5) * W, 0.0, W - 1.0))
    y1 = jnp.floor(jnp.clip((cy - h * 0.5) * H, 0.0, H - 1.0))
    x2 = jnp.floor(jnp.clip((cx + w * 0.5) * W, 0.0, W - 1.0))
    y2 = jnp.floor(jnp.clip((cy + h * 0.5) * H, 0.0, H - 1.0))
    hb, wb = y2 - y1, x2 - x1

    ioY = jax.lax.broadcasted_iota(jnp.int32, (N, H), 1).astype(jnp.float32)
    ioX = jax.lax.broadcasted_iota(jnp.int32, (N, W), 1).astype(jnp.float32)

    inv_r = 1.0 / R

    def _fdiv(a):
        return jnp.floor(a * inv_r + 1e-3)

    def _cdiv(a):
        return jnp.floor((a + (R - 1)) * inv_r + 1e-3)

    cws = []
    for j in range(R):
        xs = x1 + _fdiv(j * wb)
        xe = x1 + _cdiv((j + 1) * wb)
        inv = 1.0 / jnp.maximum(xe - xs, 1.0)
        cws.append(jnp.where((ioX >= xs) & (ioX < xe), inv, 0.0))

    ft = ftT_ref[0]
    acc = jnp.zeros((N, E), jnp.float32)
    for i in range(R):
        ys = y1 + _fdiv(i * hb)
        ye = y1 + _cdiv((i + 1) * hb)
        inv = 1.0 / jnp.maximum(ye - ys, 1.0)
        rw = jnp.where((ioY >= ys) & (ioY < ye), inv, 0.0)
        t1 = jnp.einsum('ny,ycx->ncx', rw, ft,
                        preferred_element_type=jnp.float32)
        for j in range(R):
            pooled = jnp.sum(t1 * cws[j][:, None, :], axis=-1)
            acc = acc + jnp.dot(pooled, wp_ref[0, i, j],
                                preferred_element_type=jnp.float32)

    @pl.when(k == 0)
    def _():
        acc_ref[...] = acc

    @pl.when(k > 0)
    def _():
        acc_ref[...] = acc_ref[...] + acc

    @pl.when(k == NB - 1)
    def _():
        feats = jax.nn.relu(acc_ref[...] + b_emb_ref[...])
        obj_out[0] = jnp.dot(feats, w_obj_ref[...],
                             preferred_element_type=jnp.float32) + b_obj_ref[...]
        attr_out[0] = jnp.dot(feats, w_attr_ref[...],
                              preferred_element_type=jnp.float32) + b_attr_ref[...]
        bbox_out[0] = jnp.dot(feats, w_bbox_ref[...],
                              preferred_element_type=jnp.float32) + b_bbox_ref[...]


def _rel_body(boxes_ref, cls_ref, cls_table_ref, w1a_ref, w1b_ref, w1sp_ref,
              ws1_ref, bs1_ref, ws2_ref, bs2_ref, bf1_ref, bf2_ref,
              wf2_ref, wrel_ref, brel_ref, out_ref):
    bx = boxes_ref[0]
    cls = cls_ref[0]

    TA = jnp.dot(cls_table_ref[...], w1a_ref[...],
                 preferred_element_type=jnp.float32)
    TB = jnp.dot(cls_table_ref[...], w1b_ref[...],
                 preferred_element_type=jnp.float32)
    ioK = jax.lax.broadcasted_iota(jnp.int32, (NOBJ, N), 0)
    oh = (ioK == cls).astype(jnp.float32)
    A = jax.lax.dot_general(oh, TA, (((0,), (0,)), ((), ())),
                            preferred_element_type=jnp.float32)
    Bm = jax.lax.dot_general(oh, TB, (((0,), (0,)), ((), ())),
                             preferred_element_type=jnp.float32)

    ws1 = ws1_ref[...]
    P1 = (bx[:, 0:1] * (ws1[0:1, :] + ws1[8:9, :])
          + bx[:, 1:2] * (ws1[1:2, :] + ws1[9:10, :])
          + bx[:, 2:3] * ws1[2:3, :] + bx[:, 3:4] * ws1[3:4, :])
    P2 = (bx[:, 0:1] * (ws1[4:5, :] - ws1[8:9, :])
          + bx[:, 1:2] * (ws1[5:6, :] - ws1[9:10, :])
          + bx[:, 2:3] * ws1[6:7, :] + bx[:, 3:4] * ws1[7:8, :])
    sp1 = jax.nn.relu(P1[:, None, :] + P2[None, :, :]
                      + bs1_ref[...][None])
    sp1f = sp1.reshape(N * N, 64)
    sp2 = jax.nn.relu(jnp.dot(sp1f, ws2_ref[...],
                              preferred_element_type=jnp.float32) + bs2_ref[...])
    spF = jnp.dot(sp2, w1sp_ref[...],
                  preferred_element_type=jnp.float32)

    h1 = jax.nn.relu((A[:, None, :] + Bm[None, :, :]).reshape(N * N, HID)
                     + spF + bf1_ref[...])
    h2 = jax.nn.relu(jnp.dot(h1, wf2_ref[...],
                             preferred_element_type=jnp.float32) + bf2_ref[...])
    rel = jnp.dot(h2, wrel_ref[...],
                  preferred_element_type=jnp.float32) + brel_ref[...]
    out_ref[...] = rel.reshape(1, N * N, NREL)


_s = np.repeat(np.arange(N), N)
_o = np.tile(np.arange(N), N)
_FLAT = (_s * N + _o)[_s != _o]


def kernel(features, boxes_xywh, obj_classes, w_emb, b_emb, w_obj, b_obj,
           w_attr, b_attr, w_bbox, b_bbox, cls_table, w_s1, b_s1, w_s2, b_s2,
           w_f1, b_f1, w_f2, b_f2, w_rel, b_rel):
    ftT = jnp.transpose(features, (0, 2, 1, 3))
    wp = (w_emb.reshape(NB, CB, R, R, E)
          .transpose(0, 2, 3, 1, 4))

    grid1 = (B, NB)
    obj_logits, attr_logits, bbox_pred = pl.pallas_call(
        _roi_head_body,
        grid=grid1,
        in_specs=[
            pl.BlockSpec((1, H, CB, W), lambda b, k: (b, 0, k, 0)),
            pl.BlockSpec((1, N, 4), lambda b, k: (b, 0, 0)),
            pl.BlockSpec((1, R, R, CB, E), lambda b, k: (k, 0, 0, 0, 0)),
            pl.BlockSpec((1, E), lambda b, k: (0, 0)),
            pl.BlockSpec((E, NOBJ), lambda b, k: (0, 0)),
            pl.BlockSpec((1, NOBJ), lambda b, k: (0, 0)),
            pl.BlockSpec((E, NATTR), lambda b, k: (0, 0)),
            pl.BlockSpec((1, NATTR), lambda b, k: (0, 0)),
            pl.BlockSpec((E, 4), lambda b, k: (0, 0)),
            pl.BlockSpec((1, 4), lambda b, k: (0, 0)),
        ],
        out_specs=[
            pl.BlockSpec((1, N, NOBJ), lambda b, k: (b, 0, 0)),
            pl.BlockSpec((1, N, NATTR), lambda b, k: (b, 0, 0)),
            pl.BlockSpec((1, N, 4), lambda b, k: (b, 0, 0)),
        ],
        out_shape=[
            jax.ShapeDtypeStruct((B, N, NOBJ), jnp.float32),
            jax.ShapeDtypeStruct((B, N, NATTR), jnp.float32),
            jax.ShapeDtypeStruct((B, N, 4), jnp.float32),
        ],
        scratch_shapes=[pltpu.VMEM((N, E), jnp.float32)],
        compiler_params=pltpu.CompilerParams(
            dimension_semantics=("parallel", "arbitrary"),
            vmem_limit_bytes=56 * 1024 * 1024,
        ),
    )(ftT, boxes_xywh, wp, b_emb.reshape(1, E),
      w_obj, b_obj.reshape(1, NOBJ), w_attr, b_attr.reshape(1, NATTR),
      w_bbox, b_bbox.reshape(1, 4))

    cls3 = obj_classes.astype(jnp.int32).reshape(B, 1, N)
    w1a, w1b, w1sp = w_f1[:E], w_f1[E:2 * E], w_f1[2 * E:]

    rel_full = pl.pallas_call(
        _rel_body,
        grid=(B,),
        in_specs=[
            pl.BlockSpec((1, N, 4), lambda b: (b, 0, 0)),
            pl.BlockSpec((1, 1, N), lambda b: (b, 0, 0)),
            pl.BlockSpec((NOBJ, E), lambda b: (0, 0)),
            pl.BlockSpec((E, HID), lambda b: (0, 0)),
            pl.BlockSpec((E, HID), lambda b: (0, 0)),
            pl.BlockSpec((128, HID), lambda b: (0, 0)),
            pl.BlockSpec((10, 64), lambda b: (0, 0)),
            pl.BlockSpec((1, 64), lambda b: (0, 0)),
            pl.BlockSpec((64, 128), lambda b: (0, 0)),
            pl.BlockSpec((1, 128), lambda b: (0, 0)),
            pl.BlockSpec((1, HID), lambda b: (0, 0)),
            pl.BlockSpec((1, HID), lambda b: (0, 0)),
            pl.BlockSpec((HID, HID), lambda b: (0, 0)),
            pl.BlockSpec((HID, NREL), lambda b: (0, 0)),
            pl.BlockSpec((1, NREL), lambda b: (0, 0)),
        ],
        out_specs=pl.BlockSpec((1, N * N, NREL), lambda b: (b, 0, 0)),
        out_shape=jax.ShapeDtypeStruct((B, N * N, NREL), jnp.float32),
        compiler_params=pltpu.CompilerParams(
            dimension_semantics=("parallel",),
            vmem_limit_bytes=56 * 1024 * 1024,
        ),
    )(boxes_xywh, cls3, cls_table, w1a, w1b, w1sp,
      w_s1, b_s1.reshape(1, 64), w_s2, b_s2.reshape(1, 128),
      b_f1.reshape(1, HID), b_f2.reshape(1, HID),
      w_f2, w_rel, b_rel.reshape(1, NREL))

    rel_logits = rel_full[:, _FLAT, :]
    return (obj_logits, attr_logits, bbox_pred, rel_logits)
